# baseline (speedup 1.0000x reference)
import functools

import jax
import jax.numpy as jnp
from jax import lax
from jax.experimental import pallas as pl
from jax.experimental.pallas import tpu as pltpu
from jax.experimental.pallas import tpu_sc as plsc

_NBUF = 8
_GR = 1


def _pack_body(x_ref, y_ref):
    rows = y_ref.shape[0]
    x3 = x_ref[...].reshape(rows, 4, x_ref.shape[1])
    y_ref[...] = jnp.concatenate([x3[:, u, :] for u in range(4)], axis=-1)


@functools.lru_cache(maxsize=None)
def _make_pack(n_classes, n_dim):
    c_blk = 4000
    n_blocks = n_classes // c_blk
    assert n_classes % c_blk == 0 and (c_blk * n_dim) % 128 == 0
    rows_out = c_blk * n_dim // 128
    return pl.pallas_call(
        _pack_body,
        grid=(n_blocks,),
        in_specs=[pl.BlockSpec((c_blk, n_dim), lambda i: (i, 0))],
        out_specs=pl.BlockSpec((rows_out, 128), lambda i: (i, 0)),
        out_shape=jax.ShapeDtypeStruct((n_classes * n_dim // 128, 128),
                                       jnp.float32),
    )


@functools.lru_cache(maxsize=None)
def _make_gather(batch, hist, n_dim, nc, ns):
    mesh = plsc.VectorSubcoreMesh(core_axis_name="c", subcore_axis_name="s")
    nw = nc * ns
    rows_w = batch // nw
    n_chunks = rows_w // _GR
    n_groups = n_chunks // _NBUF
    assert rows_w % _GR == 0 and n_chunks % _NBUF == 0 and n_groups >= 2

    scratch = (
        [pltpu.VMEM((rows_w, hist), jnp.int32)]
        + [pltpu.VMEM((hist, n_dim), jnp.float32) for _ in range(_NBUF)]
        + [pltpu.SemaphoreType.DMA for _ in range(2 * _NBUF)]
    )

    hp = (hist + 7) // 8 * 8
    dp = 128

    @functools.partial(
        pl.kernel,
        out_type=jax.ShapeDtypeStruct((batch, hp, dp), jnp.float32),
        mesh=mesh,
        scratch_types=scratch,
        compiler_params=pltpu.CompilerParams(use_tc_tiling_on_sc=False),
    )
    def k(idx_hbm, table_hbm, out_hbm, idx_v, *bufs_and_sems):
        rows = bufs_and_sems[:_NBUF]
        gsem = bufs_and_sems[_NBUF:2 * _NBUF]
        ssem = bufs_and_sems[2 * _NBUF:]
        wid = lax.axis_index("s") * nc + lax.axis_index("c")
        base = wid * rows_w
        pltpu.sync_copy(idx_hbm.at[pl.ds(base, rows_w)], idx_v)

        def fire_gather(j, b):
            pltpu.async_copy(table_hbm.at[idx_v.at[j]], rows[b], gsem[b])

        def wait_gather(b):
            pltpu.make_async_copy(
                table_hbm.at[idx_v.at[0]], rows[b], gsem[b]).wait()

        def fire_store(j, b):
            pltpu.async_copy(
                rows[b],
                out_hbm.at[base + j, pl.ds(0, hist), pl.ds(0, n_dim)],
                ssem[b])

        def wait_store(b):
            pltpu.make_async_copy(
                rows[b],
                out_hbm.at[base, pl.ds(0, hist), pl.ds(0, n_dim)],
                ssem[b]).wait()

        for b in range(_NBUF):
            fire_gather(b, b)

        def body(gi, carry):
            g = gi * _NBUF
            for b in range(_NBUF):
                wait_gather(b)
                fire_store(g + b, b)
            for b in range(_NBUF):
                wait_store(b)
                fire_gather(g + _NBUF + b, b)
            return carry

        lax.fori_loop(0, n_groups - 1, body, 0)

        g = (n_groups - 1) * _NBUF
        for b in range(_NBUF):
            wait_gather(b)
            fire_store(g + b, b)
        for b in range(_NBUF):
            wait_store(b)

    return k


def kernel(indices, table):
    batch, hist = indices.shape
    n_classes, n_dim = table.shape
    info = plsc.get_sparse_core_info()
    nc, ns = info.num_cores, info.num_subcores
    table_q = _make_pack(n_classes, n_dim)(table)
    table_rm = table_q.reshape(n_classes, n_dim)
    padded = _make_gather(batch, hist, n_dim, nc, ns)(indices, table_rm)
    return padded[:, :hist, :n_dim]


# speedup vs baseline: 1.2321x; 1.2321x over previous
import functools

import jax
import jax.numpy as jnp
from jax import lax
from jax.experimental import pallas as pl
from jax.experimental.pallas import tpu as pltpu
from jax.experimental.pallas import tpu_sc as plsc

_NBUF = 8
_GR = 1


@functools.lru_cache(maxsize=None)
def _make_gather(batch, hist, n_dim, nc, ns):
    mesh = plsc.VectorSubcoreMesh(core_axis_name="c", subcore_axis_name="s")
    nw = nc * ns
    rows_w = batch // nw
    n_chunks = rows_w // _GR
    n_groups = n_chunks // _NBUF
    assert rows_w % _GR == 0 and n_chunks % _NBUF == 0 and n_groups >= 2

    scratch = (
        [pltpu.VMEM((rows_w, hist), jnp.int32)]
        + [pltpu.VMEM((hist, n_dim), jnp.float32) for _ in range(_NBUF)]
        + [pltpu.SemaphoreType.DMA for _ in range(2 * _NBUF)]
    )

    hp = (hist + 7) // 8 * 8
    dp = 128

    @functools.partial(
        pl.kernel,
        out_type=jax.ShapeDtypeStruct((batch, hp, dp), jnp.float32),
        mesh=mesh,
        scratch_types=scratch,
        compiler_params=pltpu.CompilerParams(use_tc_tiling_on_sc=False),
    )
    def k(idx_hbm, table_hbm, out_hbm, idx_v, *bufs_and_sems):
        rows = bufs_and_sems[:_NBUF]
        gsem = bufs_and_sems[_NBUF:2 * _NBUF]
        ssem = bufs_and_sems[2 * _NBUF:]
        wid = lax.axis_index("s") * nc + lax.axis_index("c")
        base = wid * rows_w
        pltpu.sync_copy(idx_hbm.at[pl.ds(base, rows_w)], idx_v)

        def fire_gather(j, b):
            pltpu.async_copy(table_hbm.at[idx_v.at[j]], rows[b], gsem[b])

        def wait_gather(b):
            pltpu.make_async_copy(
                table_hbm.at[idx_v.at[0]], rows[b], gsem[b]).wait()

        def fire_store(j, b):
            pltpu.async_copy(
                rows[b],
                out_hbm.at[base + j, pl.ds(0, hist), pl.ds(0, n_dim)],
                ssem[b])

        def wait_store(b):
            pltpu.make_async_copy(
                rows[b],
                out_hbm.at[base, pl.ds(0, hist), pl.ds(0, n_dim)],
                ssem[b]).wait()

        for b in range(_NBUF):
            fire_gather(b, b)

        def body(gi, carry):
            g = gi * _NBUF
            for b in range(_NBUF):
                wait_gather(b)
                fire_store(g + b, b)
            for b in range(_NBUF):
                wait_store(b)
                fire_gather(g + _NBUF + b, b)
            return carry

        lax.fori_loop(0, n_groups - 1, body, 0)

        g = (n_groups - 1) * _NBUF
        for b in range(_NBUF):
            wait_gather(b)
            fire_store(g + b, b)
        for b in range(_NBUF):
            wait_store(b)

    return k


def kernel(indices, table):
    batch, hist = indices.shape
    n_classes, n_dim = table.shape
    info = plsc.get_sparse_core_info()
    nc, ns = info.num_cores, info.num_subcores
    padded = _make_gather(batch, hist, n_dim, nc, ns)(indices, table)
    return padded[:, :hist, :n_dim]


# speedup vs baseline: 1.2618x; 1.0241x over previous
import functools

import jax
import jax.numpy as jnp
from jax import lax
from jax.experimental import pallas as pl
from jax.experimental.pallas import tpu as pltpu
from jax.experimental.pallas import tpu_sc as plsc

_NBUF = 8
_GR = 1


@functools.lru_cache(maxsize=None)
def _make_gather(batch, hist, n_dim, nc, ns, pair):
    mesh = plsc.VectorSubcoreMesh(core_axis_name="c", subcore_axis_name="s")
    nw = nc * ns
    rows_w = batch // nw
    n_chunks = rows_w // pair
    n_groups = n_chunks // _NBUF
    assert rows_w % pair == 0 and n_chunks % _NBUF == 0 and n_groups >= 2

    scratch = (
        [pltpu.VMEM((n_chunks, pair * hist), jnp.int32)]
        + [pltpu.VMEM((pair * hist, n_dim), jnp.float32)
           for _ in range(_NBUF)]
        + [pltpu.SemaphoreType.DMA for _ in range(2 * _NBUF)]
    )

    hp = (hist + 7) // 8 * 8
    dp = 128

    @functools.partial(
        pl.kernel,
        out_type=jax.ShapeDtypeStruct((batch, hp, dp), jnp.float32),
        mesh=mesh,
        scratch_types=scratch,
        compiler_params=pltpu.CompilerParams(use_tc_tiling_on_sc=False),
    )
    def k(idx_hbm, table_hbm, out_hbm, idx_v, *bufs_and_sems):
        rows = bufs_and_sems[:_NBUF]
        gsem = bufs_and_sems[_NBUF:2 * _NBUF]
        ssem = bufs_and_sems[2 * _NBUF:]
        wid = lax.axis_index("s") * nc + lax.axis_index("c")
        base = wid * rows_w
        pltpu.sync_copy(idx_hbm.at[pl.ds(wid * n_chunks, n_chunks)], idx_v)

        def fire_gather(j, b):
            pltpu.async_copy(table_hbm.at[idx_v.at[j]], rows[b], gsem[b])

        def wait_gather(b):
            pltpu.make_async_copy(
                table_hbm.at[idx_v.at[0]], rows[b], gsem[b]).wait()

        def fire_store(j, b):
            for u in range(pair):
                pltpu.async_copy(
                    rows[b].at[pl.ds(u * hist, hist)],
                    out_hbm.at[base + j * pair + u,
                               pl.ds(0, hist), pl.ds(0, n_dim)],
                    ssem[b])

        def wait_store(b):
            for u in range(pair):
                pltpu.make_async_copy(
                    rows[b].at[pl.ds(u * hist, hist)],
                    out_hbm.at[base + u, pl.ds(0, hist), pl.ds(0, n_dim)],
                    ssem[b]).wait()

        for b in range(_NBUF):
            fire_gather(b, b)

        def body(gi, carry):
            g = gi * _NBUF
            for b in range(_NBUF):
                wait_gather(b)
                fire_store(g + b, b)
            for b in range(_NBUF):
                wait_store(b)
                fire_gather(g + _NBUF + b, b)
            return carry

        lax.fori_loop(0, n_groups - 1, body, 0)

        g = (n_groups - 1) * _NBUF
        for b in range(_NBUF):
            wait_gather(b)
            fire_store(g + b, b)
        for b in range(_NBUF):
            wait_store(b)

    return k


def kernel(indices, table):
    batch, hist = indices.shape
    n_classes, n_dim = table.shape
    info = plsc.get_sparse_core_info()
    nc, ns = info.num_cores, info.num_subcores
    pair = 2 if (batch % 2 == 0 and 2 * hist <= 128) else 1
    idx = indices.reshape(batch // pair, pair * hist)
    padded = _make_gather(batch, hist, n_dim, nc, ns, pair)(idx, table)
    return padded[:, :hist, :n_dim]
